# baseline (speedup 1.0000x reference)
import jax
import jax.numpy as jnp
from jax.experimental import pallas as pl
from jax.experimental.pallas import tpu as pltpu


def _make_kernel(NR):
    def _fused(x_ref, wr_ref, br_ref, we_ref, be_ref, out_ref, acc_ref, v_ref):
        i = pl.program_id(0)

        @pl.when(i < NR)
        def _():
            xb = x_ref[...]
            logits = jax.lax.dot_general(
                xb, wr_ref[...], (((1,), (1,)), ((), ())),
                preferred_element_type=jnp.float32)
            logits = logits + br_ref[...]
            m = jnp.max(logits, axis=1, keepdims=True)
            denom = jnp.sum(jnp.exp(logits - m), axis=1, keepdims=True)
            G = 1.0 / denom
            u = jax.lax.dot_general(
                G, xb, (((0,), (0,)), ((), ())),
                preferred_element_type=jnp.float32)
            gsum = jnp.sum(G, axis=0, keepdims=True)
            part = jnp.concatenate(
                [u, jnp.broadcast_to(gsum, u.shape)], axis=0)

            @pl.when(i == 0)
            def _():
                acc_ref[...] = part

            @pl.when(i != 0)
            def _():
                acc_ref[...] += part

        @pl.when(i == NR)
        def _():
            u = acc_ref[0:1, :]
            g = acc_ref[1, 0]
            v = jax.lax.dot_general(
                u, we_ref[...], (((1,), (1,)), ((), ())),
                preferred_element_type=jnp.float32)
            v_ref[...] = v + g * be_ref[...]

        @pl.when(i >= NR)
        def _():
            out_ref[...] = jnp.broadcast_to(v_ref[...], out_ref.shape)

    return _fused


def kernel(x, W_r, b_r, W_e, b_e):
    b, s, h = x.shape
    bs = b * s
    e = W_r.shape[0]
    xf = x.reshape(bs, h)
    br2 = b_r.reshape(1, e)
    be2 = b_e.reshape(1, h)

    TB = 512
    RB = 1024
    NR = bs // TB
    NW = bs // RB

    yflat = pl.pallas_call(
        _make_kernel(NR),
        grid=(NR + NW,),
        in_specs=[
            pl.BlockSpec((TB, h), lambda i: (jnp.minimum(i, NR - 1), 0)),
            pl.BlockSpec((e, h), lambda i: (0, 0)),
            pl.BlockSpec((1, e), lambda i: (0, 0)),
            pl.BlockSpec((h, h), lambda i: (0, 0)),
            pl.BlockSpec((1, h), lambda i: (0, 0)),
        ],
        out_specs=pl.BlockSpec((RB, h), lambda i: (jnp.maximum(i - NR, 0), 0)),
        out_shape=jax.ShapeDtypeStruct((bs, h), jnp.float32),
        scratch_shapes=[
            pltpu.VMEM((2, h), jnp.float32),
            pltpu.VMEM((1, h), jnp.float32),
        ],
        compiler_params=pltpu.CompilerParams(
            dimension_semantics=("arbitrary",)),
    )(xf, W_r, br2, W_e, be2)

    return yflat.reshape(b, s, h)


# speedup vs baseline: 1.0695x; 1.0695x over previous
import jax
import jax.numpy as jnp
from jax.experimental import pallas as pl
from jax.experimental.pallas import tpu as pltpu


def _make_kernel(NR):
    def _fused(x_ref, wr_ref, br_ref, we_ref, be_ref, out_ref, acc_ref, v_ref):
        i = pl.program_id(0)

        @pl.when(i < NR)
        def _():
            xb = x_ref[...]
            logits = jax.lax.dot_general(
                xb, wr_ref[...], (((1,), (1,)), ((), ())),
                preferred_element_type=jnp.float32)
            logits = logits + br_ref[...]
            m = jnp.max(logits, axis=1, keepdims=True)
            denom = jnp.sum(jnp.exp(logits - m), axis=1, keepdims=True)
            G = 1.0 / denom
            u = jax.lax.dot_general(
                G, xb, (((0,), (0,)), ((), ())),
                preferred_element_type=jnp.float32)
            gsum = jnp.sum(G, axis=0, keepdims=True)
            part = jnp.concatenate(
                [u, jnp.broadcast_to(gsum, u.shape)], axis=0)

            @pl.when(i == 0)
            def _():
                acc_ref[...] = part

            @pl.when(i != 0)
            def _():
                acc_ref[...] += part

        @pl.when(i == NR)
        def _():
            u = acc_ref[0:1, :]
            g = acc_ref[1, 0]
            v = jax.lax.dot_general(
                u, we_ref[...], (((1,), (1,)), ((), ())),
                preferred_element_type=jnp.float32)
            v_ref[...] = v + g * be_ref[...]

        @pl.when(i >= NR)
        def _():
            out_ref[...] = jnp.broadcast_to(v_ref[...], out_ref.shape)

    return _fused


def kernel(x, W_r, b_r, W_e, b_e):
    b, s, h = x.shape
    bs = b * s
    e = W_r.shape[0]
    xf = x.reshape(bs, h)
    br2 = b_r.reshape(1, e)
    be2 = b_e.reshape(1, h)

    TB = 1024
    RB = 1024
    NR = bs // TB
    NW = bs // RB

    yflat = pl.pallas_call(
        _make_kernel(NR),
        grid=(NR + NW,),
        in_specs=[
            pl.BlockSpec((TB, h), lambda i: (jnp.minimum(i, NR - 1), 0)),
            pl.BlockSpec((e, h), lambda i: (0, 0)),
            pl.BlockSpec((1, e), lambda i: (0, 0)),
            pl.BlockSpec((h, h), lambda i: (0, 0)),
            pl.BlockSpec((1, h), lambda i: (0, 0)),
        ],
        out_specs=pl.BlockSpec((RB, h), lambda i: (jnp.maximum(i - NR, 0), 0)),
        out_shape=jax.ShapeDtypeStruct((bs, h), jnp.float32),
        scratch_shapes=[
            pltpu.VMEM((2, h), jnp.float32),
            pltpu.VMEM((1, h), jnp.float32),
        ],
        compiler_params=pltpu.CompilerParams(
            dimension_semantics=("arbitrary",)),
    )(xf, W_r, br2, W_e, be2)

    return yflat.reshape(b, s, h)


# speedup vs baseline: 1.0930x; 1.0219x over previous
import jax
import jax.numpy as jnp
from jax.experimental import pallas as pl
from jax.experimental.pallas import tpu as pltpu


def _make_kernel(NR, NW, RB):
    def _fused(x_ref, wr_ref, br_ref, we_ref, be_ref, out_ref,
               acc_ref, buf_ref, sems):
        i = pl.program_id(0)

        @pl.when(i < NR)
        def _():
            xb = x_ref[...]
            logits = jax.lax.dot_general(
                xb, wr_ref[...], (((1,), (1,)), ((), ())),
                preferred_element_type=jnp.float32)
            logits = logits + br_ref[...]
            m = jnp.max(logits, axis=1, keepdims=True)
            denom = jnp.sum(jnp.exp(logits - m), axis=1, keepdims=True)
            G = 1.0 / denom
            u = jax.lax.dot_general(
                G, xb, (((0,), (0,)), ((), ())),
                preferred_element_type=jnp.float32)
            gsum = jnp.sum(G, axis=0, keepdims=True)
            part = jnp.concatenate(
                [u, jnp.broadcast_to(gsum, u.shape)], axis=0)

            @pl.when(i == 0)
            def _():
                acc_ref[...] = part

            @pl.when(i != 0)
            def _():
                acc_ref[...] += part

        @pl.when(i == NR)
        def _():
            u = acc_ref[0:1, :]
            g = acc_ref[1, 0]
            v = jax.lax.dot_general(
                u, we_ref[...], (((1,), (1,)), ((), ())),
                preferred_element_type=jnp.float32)
            v = v + g * be_ref[...]
            buf_ref[...] = jnp.broadcast_to(v, buf_ref.shape)
            copies = []
            for r in range(NW):
                c = pltpu.make_async_copy(
                    buf_ref, out_ref.at[pl.ds(r * RB, RB), :], sems.at[r])
                c.start()
                copies.append(c)
            for c in copies:
                c.wait()

    return _fused


def kernel(x, W_r, b_r, W_e, b_e):
    b, s, h = x.shape
    bs = b * s
    e = W_r.shape[0]
    xf = x.reshape(bs, h)
    br2 = b_r.reshape(1, e)
    be2 = b_e.reshape(1, h)

    TB = 1024
    RB = 1024
    NR = bs // TB
    NW = bs // RB

    yflat = pl.pallas_call(
        _make_kernel(NR, NW, RB),
        grid=(NR + 1,),
        in_specs=[
            pl.BlockSpec((TB, h), lambda i: (jnp.minimum(i, NR - 1), 0)),
            pl.BlockSpec((e, h), lambda i: (0, 0)),
            pl.BlockSpec((1, e), lambda i: (0, 0)),
            pl.BlockSpec((h, h), lambda i: (0, 0)),
            pl.BlockSpec((1, h), lambda i: (0, 0)),
        ],
        out_specs=pl.BlockSpec(memory_space=pl.ANY),
        out_shape=jax.ShapeDtypeStruct((bs, h), jnp.float32),
        scratch_shapes=[
            pltpu.VMEM((2, h), jnp.float32),
            pltpu.VMEM((RB, h), jnp.float32),
            pltpu.SemaphoreType.DMA((bs // RB,)),
        ],
        compiler_params=pltpu.CompilerParams(
            dimension_semantics=("arbitrary",)),
    )(xf, W_r, br2, W_e, be2)

    return yflat.reshape(b, s, h)


# speedup vs baseline: 1.1006x; 1.0070x over previous
import jax
import jax.numpy as jnp
from jax.experimental import pallas as pl
from jax.experimental.pallas import tpu as pltpu


def _make_kernel(NR, NW, RB):
    def _fused(x_ref, wr_ref, br_ref, we_ref, be_ref, out_ref,
               acc_ref, buf_ref, sems):
        i = pl.program_id(0)

        @pl.when(i < NR)
        def _():
            xb = x_ref[...]
            logits = jax.lax.dot_general(
                xb, wr_ref[...], (((1,), (1,)), ((), ())),
                preferred_element_type=jnp.float32)
            logits = logits + br_ref[...]
            m = jnp.max(logits, axis=1, keepdims=True)
            denom = jnp.sum(jnp.exp(logits - m), axis=1, keepdims=True)
            G = 1.0 / denom
            u = jax.lax.dot_general(
                G, xb, (((0,), (0,)), ((), ())),
                preferred_element_type=jnp.float32)
            gsum = jnp.sum(G, axis=0, keepdims=True)
            part = jnp.concatenate(
                [u, jnp.broadcast_to(gsum, u.shape)], axis=0)

            @pl.when(i == 0)
            def _():
                acc_ref[...] = part

            @pl.when(i != 0)
            def _():
                acc_ref[...] += part

        @pl.when(i == NR)
        def _():
            u = acc_ref[0:1, :]
            g = acc_ref[1, 0]
            v = jax.lax.dot_general(
                u, we_ref[...], (((1,), (1,)), ((), ())),
                preferred_element_type=jnp.float32)
            v = v + g * be_ref[...]
            buf_ref[...] = jnp.broadcast_to(v, buf_ref.shape)
            copies = []
            for r in range(NW):
                c = pltpu.make_async_copy(
                    buf_ref, out_ref.at[pl.ds(r * RB, RB), :], sems.at[r])
                c.start()
                copies.append(c)
            for c in copies:
                c.wait()

    return _fused


def kernel(x, W_r, b_r, W_e, b_e):
    b, s, h = x.shape
    bs = b * s
    e = W_r.shape[0]
    xf = x.reshape(bs, h)
    br2 = b_r.reshape(1, e)
    be2 = b_e.reshape(1, h)

    TB = 2048
    RB = 512
    NR = bs // TB
    NW = bs // RB

    yflat = pl.pallas_call(
        _make_kernel(NR, NW, RB),
        grid=(NR + 1,),
        in_specs=[
            pl.BlockSpec((TB, h), lambda i: (jnp.minimum(i, NR - 1), 0)),
            pl.BlockSpec((e, h), lambda i: (0, 0)),
            pl.BlockSpec((1, e), lambda i: (0, 0)),
            pl.BlockSpec((h, h), lambda i: (0, 0)),
            pl.BlockSpec((1, h), lambda i: (0, 0)),
        ],
        out_specs=pl.BlockSpec(memory_space=pl.ANY),
        out_shape=jax.ShapeDtypeStruct((bs, h), jnp.float32),
        scratch_shapes=[
            pltpu.VMEM((2, h), jnp.float32),
            pltpu.VMEM((RB, h), jnp.float32),
            pltpu.SemaphoreType.DMA((bs // RB,)),
        ],
        compiler_params=pltpu.CompilerParams(
            dimension_semantics=("arbitrary",)),
    )(xf, W_r, br2, W_e, be2)

    return yflat.reshape(b, s, h)
